# Initial kernel scaffold; baseline (speedup 1.0000x reference)
import jax
import jax.numpy as jnp
from jax.experimental import pallas as pl


def kernel(x, batch, W1, b1, W2, b2, Wt, bt):
    raise NotImplementedError("write your pallas kernel here")


# speedup vs baseline: 13.5572x; 13.5572x over previous
import functools

import jax
import jax.numpy as jnp
from jax import lax
from jax.experimental import pallas as pl
from jax.experimental.pallas import tpu as pltpu

_G = 256
_BN = 1000


def _body(batch_ref, x_ref, W1_ref, b1_ref, W2_ref, b2_ref, Wt_ref, bt_ref,
          out_ref, m_ref, z_ref, p_ref):
    i = pl.program_id(0)
    nb = pl.num_programs(0)

    @pl.when(i == 0)
    def _init():
        m_ref[...] = jnp.zeros_like(m_ref)
        z_ref[...] = jnp.zeros_like(z_ref)
        p_ref[...] = jnp.zeros_like(p_ref)

    xb = x_ref[...]
    h = jnp.tanh(
        jnp.dot(xb, W1_ref[...], preferred_element_type=jnp.float32)
        + b1_ref[...])
    sT = lax.dot_general(W2_ref[...], h, (((0,), (1,)), ((), ())),
                         preferred_element_type=jnp.float32)
    sT = sT + b2_ref[0, 0]

    ids = batch_ref[0]
    gi = lax.broadcasted_iota(jnp.int32, (_G, sT.shape[1]), 0)
    M = gi == ids

    neg = jnp.float32(-1e30)
    bm = jnp.max(jnp.where(M, sT, neg), axis=1, keepdims=True)
    m_old = m_ref[...]
    m_new = jnp.maximum(m_old, bm)
    alpha = jnp.exp(m_old - m_new)
    mpn = jnp.sum(jnp.where(M, m_new, 0.0), axis=0, keepdims=True)
    eT = jnp.exp(sT - mpn)
    E = jnp.where(M, eT, 0.0)
    z_ref[...] = z_ref[...] * alpha + jnp.sum(E, axis=1, keepdims=True)
    p_ref[...] = p_ref[...] * alpha + jnp.dot(
        E, xb, preferred_element_type=jnp.float32)
    m_ref[...] = m_new

    @pl.when(i == nb - 1)
    def _fin():
        z = z_ref[...]
        pooled = jnp.where(z > 0.0, p_ref[...] / z, 0.0)
        out_ref[...] = jnp.dot(
            pooled, Wt_ref[...], preferred_element_type=jnp.float32
        ) + bt_ref[...]


@jax.jit
def kernel(x, batch, W1, b1, W2, b2, Wt, bt):
    N, D = x.shape
    DH = W1.shape[1]
    DO = Wt.shape[1]

    nb = -(-N // _BN)
    Np = nb * _BN
    if Np != N:
        x = jnp.pad(x, ((0, Np - N), (0, 0)))
        batch = jnp.pad(batch.astype(jnp.int32), (0, Np - N),
                        constant_values=_G)
    batch3 = batch.astype(jnp.int32).reshape(nb, 1, _BN)
    b1r = b1.reshape(1, DH).astype(jnp.float32)
    b2r = b2.reshape(1, 1).astype(jnp.float32)
    btr = bt.reshape(1, DO).astype(jnp.float32)

    out = pl.pallas_call(
        _body,
        grid=(nb,),
        in_specs=[
            pl.BlockSpec((1, 1, _BN), lambda i: (i, 0, 0)),
            pl.BlockSpec((_BN, D), lambda i: (i, 0)),
            pl.BlockSpec((D, DH), lambda i: (0, 0)),
            pl.BlockSpec((1, DH), lambda i: (0, 0)),
            pl.BlockSpec((DH, 1), lambda i: (0, 0)),
            pl.BlockSpec((1, 1), lambda i: (0, 0)),
            pl.BlockSpec((D, DO), lambda i: (0, 0)),
            pl.BlockSpec((1, DO), lambda i: (0, 0)),
        ],
        out_specs=pl.BlockSpec((_G, DO), lambda i: (0, 0)),
        out_shape=jax.ShapeDtypeStruct((_G, DO), jnp.float32),
        scratch_shapes=[
            pltpu.VMEM((_G, 1), jnp.float32),
            pltpu.VMEM((_G, 1), jnp.float32),
            pltpu.VMEM((_G, D), jnp.float32),
        ],
        compiler_params=pltpu.CompilerParams(
            dimension_semantics=("arbitrary",)),
    )(batch3, x, W1, b1r, W2, b2r, Wt, btr)
    return out


# speedup vs baseline: 16.8097x; 1.2399x over previous
import functools

import jax
import jax.numpy as jnp
from jax import lax
from jax.experimental import pallas as pl
from jax.experimental.pallas import tpu as pltpu

_G = 256
_BN = 2000


def _body(batch_ref, x_ref, W1_ref, b1_ref, W2_ref, b2_ref, Wt_ref, bt_ref,
          out_ref, m_ref, z_ref, p_ref):
    i = pl.program_id(0)
    nb = pl.num_programs(0)

    @pl.when(i == 0)
    def _init():
        m_ref[...] = jnp.zeros_like(m_ref)
        z_ref[...] = jnp.zeros_like(z_ref)
        p_ref[...] = jnp.zeros_like(p_ref)

    xb = x_ref[...]
    xb_bf = xb.astype(jnp.bfloat16)
    h = jnp.tanh(
        jnp.dot(xb_bf, W1_ref[...].astype(jnp.bfloat16),
                preferred_element_type=jnp.float32)
        + b1_ref[...])
    sT = lax.dot_general(W2_ref[...], h, (((0,), (1,)), ((), ())),
                         preferred_element_type=jnp.float32)
    sT = sT + b2_ref[0, 0]

    ids = batch_ref[0]
    gi = lax.broadcasted_iota(jnp.int32, (_G, sT.shape[1]), 0)
    M = gi == ids

    neg = jnp.float32(-1e30)
    bm = jnp.max(jnp.where(M, sT, neg), axis=1, keepdims=True)
    m_old = m_ref[...]
    m_new = jnp.maximum(m_old, bm)
    alpha = jnp.exp(m_old - m_new)
    mpn = jnp.sum(jnp.where(M, m_new, 0.0), axis=0, keepdims=True)
    eT = jnp.exp(sT - mpn)
    E = jnp.where(M, eT, 0.0)
    z_ref[...] = z_ref[...] * alpha + jnp.sum(E, axis=1, keepdims=True)
    p_ref[...] = p_ref[...] * alpha + jnp.dot(
        E.astype(jnp.bfloat16), xb_bf,
        preferred_element_type=jnp.float32)
    m_ref[...] = m_new

    @pl.when(i == nb - 1)
    def _fin():
        z = z_ref[...]
        pooled = jnp.where(z > 0.0, p_ref[...] / z, 0.0)
        out_ref[...] = jnp.dot(
            pooled, Wt_ref[...], preferred_element_type=jnp.float32
        ) + bt_ref[...]


@jax.jit
def kernel(x, batch, W1, b1, W2, b2, Wt, bt):
    N, D = x.shape
    DH = W1.shape[1]
    DO = Wt.shape[1]

    nb = -(-N // _BN)
    Np = nb * _BN
    if Np != N:
        x = jnp.pad(x, ((0, Np - N), (0, 0)))
        batch = jnp.pad(batch.astype(jnp.int32), (0, Np - N),
                        constant_values=_G)
    batch3 = batch.astype(jnp.int32).reshape(nb, 1, _BN)
    b1r = b1.reshape(1, DH).astype(jnp.float32)
    b2r = b2.reshape(1, 1).astype(jnp.float32)
    btr = bt.reshape(1, DO).astype(jnp.float32)

    out = pl.pallas_call(
        _body,
        grid=(nb,),
        in_specs=[
            pl.BlockSpec((1, 1, _BN), lambda i: (i, 0, 0)),
            pl.BlockSpec((_BN, D), lambda i: (i, 0)),
            pl.BlockSpec((D, DH), lambda i: (0, 0)),
            pl.BlockSpec((1, DH), lambda i: (0, 0)),
            pl.BlockSpec((DH, 1), lambda i: (0, 0)),
            pl.BlockSpec((1, 1), lambda i: (0, 0)),
            pl.BlockSpec((D, DO), lambda i: (0, 0)),
            pl.BlockSpec((1, DO), lambda i: (0, 0)),
        ],
        out_specs=pl.BlockSpec((_G, DO), lambda i: (0, 0)),
        out_shape=jax.ShapeDtypeStruct((_G, DO), jnp.float32),
        scratch_shapes=[
            pltpu.VMEM((_G, 1), jnp.float32),
            pltpu.VMEM((_G, 1), jnp.float32),
            pltpu.VMEM((_G, D), jnp.float32),
        ],
        compiler_params=pltpu.CompilerParams(
            dimension_semantics=("arbitrary",)),
    )(batch3, x, W1, b1r, W2, b2r, Wt, btr)
    return out
